# baseline (speedup 1.0000x reference)
import functools

import jax
import jax.numpy as jnp
import numpy as np
from jax import lax
from jax.experimental import pallas as pl
from jax.experimental.pallas import tpu as pltpu
from jax.experimental.pallas import tpu_sc as plsc

_N = 20000
_M = 100
_NPAD = 20480
_ROWS = _NPAD // 128
_K = 256
_SENT = np.int32(1 << 20)

def _rank_desc(v):
    order = np.argsort(-v, kind="stable")
    r = np.empty(_N, np.int32)
    r[order] = np.arange(_N, dtype=np.int32)
    return r


def _threefry2x32(k0, k1, x0, x1):
    rot = ((13, 15, 26, 6), (17, 29, 16, 24))
    ks = (np.uint32(k0), np.uint32(k1),
          np.uint32(np.uint32(k0) ^ np.uint32(k1) ^ np.uint32(0x1BD11BDA)))
    x0 = (x0 + ks[0]).astype(np.uint32)
    x1 = (x1 + ks[1]).astype(np.uint32)
    for i in range(5):
        for r in rot[i % 2]:
            x0 = (x0 + x1).astype(np.uint32)
            x1 = ((x1 << np.uint32(r)) | (x1 >> np.uint32(32 - r))).astype(np.uint32)
            x1 = (x1 ^ x0).astype(np.uint32)
        x0 = (x0 + ks[(i + 1) % 3]).astype(np.uint32)
        x1 = (x1 + ks[(i + 2) % 3] + np.uint32(i + 1)).astype(np.uint32)
    return x0, x1


def _draw_tiebreak():
    x0, x1 = _threefry2x32(0, 42, np.zeros(_N, np.uint32),
                           np.arange(_N, dtype=np.uint32))
    bits = (x0 ^ x1).astype(np.uint32)
    f = ((bits >> np.uint32(9)) | np.uint32(0x3F800000)).view(np.float32)
    u = (f - np.float32(1.0)) * np.float32(0.001) + np.float32(0.0)
    return np.maximum(np.float32(0.0), u)


def _build_key_bands():
    rnd = _draw_tiebreak()
    kpos = _rank_desc((np.float32(1.0) + rnd).astype(np.float32))
    kneg = _rank_desc((np.float32(0.5) + rnd).astype(np.float32)) + np.int32(_N)
    kinv = np.arange(_N, dtype=np.int32) + np.int32(2 * _N)

    def pad_band(b):
        return np.concatenate([b, np.full(_NPAD - _N, _SENT, np.int32)])

    return np.stack(
        [pad_band(kpos), pad_band(kneg), pad_band(kinv)]).reshape(3, _ROWS, 128)


_KBANDS = _build_key_bands()


def _key_bands():
    return _KBANDS


_CHUNK_ROWS = 32


def _tc_match_body(a_ref, gt_ref, kb_ref, box_ref, pos_ref, key_ref):
    for ch in range(_ROWS // _CHUNK_ROWS):
        sl = slice(ch * _CHUNK_ROWS, (ch + 1) * _CHUNK_ROWS)
        ay0 = a_ref[0, sl, :]
        ax0 = a_ref[1, sl, :]
        ay1 = a_ref[2, sl, :]
        ax1 = a_ref[3, sl, :]
        area_a = (ay1 - ay0) * (ax1 - ax0)

        def gt_step(j, carry, ay0=ay0, ax0=ax0, ay1=ay1, ax1=ax1,
                    area_a=area_a):
            bn, bd, bj = carry
            gy0 = gt_ref[j, 0]
            gx0 = gt_ref[j, 1]
            gy1 = gt_ref[j, 2]
            gx1 = gt_ref[j, 3]
            iy0 = jnp.maximum(ay0, gy0)
            ix0 = jnp.maximum(ax0, gx0)
            iy1 = jnp.minimum(ay1, gy1)
            ix1 = jnp.minimum(ax1, gx1)
            ih = jnp.maximum(iy1 - iy0, jnp.float32(0.0))
            iw = jnp.maximum(ix1 - ix0, jnp.float32(0.0))
            inter = ih * iw
            area_g = (gy1 - gy0) * (gx1 - gx0)
            union = (area_a + area_g) - inter
            upd = inter * bd > bn * union
            bn = jnp.where(upd, inter, bn)
            bd = jnp.where(upd, union, bd)
            bj = jnp.where(upd, j, bj)
            return bn, bd, bj

        shp = (_CHUNK_ROWS, 128)
        neg1 = jnp.full(shp, -1.0, jnp.float32)
        one = jnp.ones(shp, jnp.float32)
        zeroi = jnp.zeros(shp, jnp.int32)
        bn, bd, bj = lax.fori_loop(0, _M, gt_step, (neg1, one, zeroi))

        def coord_step(j, carry, bj=bj):
            by0, bx0, by1, bx1 = carry
            m = bj == j
            by0 = jnp.where(m, gt_ref[j, 0], by0)
            bx0 = jnp.where(m, gt_ref[j, 1], bx0)
            by1 = jnp.where(m, gt_ref[j, 2], by1)
            bx1 = jnp.where(m, gt_ref[j, 3], bx1)
            return by0, bx0, by1, bx1

        zero = jnp.zeros(shp, jnp.float32)
        by0, bx0, by1, bx1 = lax.fori_loop(
            0, _M, coord_step, (zero, zero, zero, zero))

        best = bn / bd
        pos = best >= jnp.float32(0.7)
        neg = best < jnp.float32(0.3)
        pos_ref[sl, :] = jnp.where(pos, jnp.float32(1.0), jnp.float32(0.0))

        box_ref[0, sl, :] = by0
        box_ref[1, sl, :] = bx0
        box_ref[2, sl, :] = by1
        box_ref[3, sl, :] = bx1

        key_ref[sl, :] = jnp.where(
            pos, kb_ref[0, sl, :],
            jnp.where(neg, kb_ref[1, sl, :], kb_ref[2, sl, :]))


def _tc_encode_body(a_ref, box_ref, enc_ref):
    ay0 = a_ref[0]
    ax0 = a_ref[1]
    ay1 = a_ref[2]
    ax1 = a_ref[3]
    by0 = box_ref[0]
    bx0 = box_ref[1]
    by1 = box_ref[2]
    bx1 = box_ref[3]
    ah = ay1 - ay0
    aw = ax1 - ax0
    acy = ay0 + jnp.float32(0.5) * ah
    acx = ax0 + jnp.float32(0.5) * aw
    bh = by1 - by0
    bw = bx1 - bx0
    bcy = by0 + jnp.float32(0.5) * bh
    bcx = bx0 + jnp.float32(0.5) * bw
    enc_ref[0] = ((bcy - acy) / ah) / jnp.float32(0.1)
    enc_ref[1] = ((bcx - acx) / aw) / jnp.float32(0.1)
    enc_ref[2] = jnp.log(bh / ah) / jnp.float32(0.2)
    enc_ref[3] = jnp.log(bw / aw) / jnp.float32(0.2)


def _run_tc_match(a_t, gt_boxes, kbands):
    f32 = jnp.float32
    return pl.pallas_call(
        _tc_match_body,
        out_shape=[
            jax.ShapeDtypeStruct((4, _ROWS, 128), f32),
            jax.ShapeDtypeStruct((_ROWS, 128), f32),
            jax.ShapeDtypeStruct((_ROWS, 128), jnp.int32),
        ],
        in_specs=[
            pl.BlockSpec(memory_space=pltpu.VMEM),
            pl.BlockSpec(memory_space=pltpu.SMEM),
            pl.BlockSpec(memory_space=pltpu.VMEM),
        ],
        out_specs=[
            pl.BlockSpec(memory_space=pltpu.VMEM),
            pl.BlockSpec(memory_space=pltpu.VMEM),
            pl.BlockSpec(memory_space=pltpu.VMEM),
        ],
    )(a_t, gt_boxes, kbands)


def _run_tc_encode(a_t, boxes):
    return pl.pallas_call(
        _tc_encode_body,
        out_shape=jax.ShapeDtypeStruct((4, _ROWS, 128), jnp.float32),
        in_specs=[
            pl.BlockSpec(memory_space=pltpu.VMEM),
            pl.BlockSpec(memory_space=pltpu.VMEM),
        ],
        out_specs=pl.BlockSpec(memory_space=pltpu.VMEM),
    )(a_t, boxes)


_SC_CHUNK = _NPAD // 16
_SC_VREGS = _SC_CHUNK // 16
_STEPS = (4096, 256, 16, 1)


_GATHER_DNUMS = lax.GatherDimensionNumbers(
    offset_dims=(), collapsed_slice_dims=(0,), start_index_map=(0,))


def _perm16(x, idx):
    return lax.gather(x, idx.reshape(16, 1), _GATHER_DNUMS, (1,),
                      mode=lax.GatherScatterMode.PROMISE_IN_BOUNDS)


def _lane_sum(x, lane):
    for sh in (1, 2, 4, 8):
        x = x + _perm16(x, jnp.bitwise_and(lane + jnp.int32(sh), jnp.int32(15)))
    return x


def _sc_sample_body(keys_hbm, out_hbm, kv, ov, cnt_ref, allc_ref, shared):
    c = lax.axis_index("c")
    s = lax.axis_index("s")
    base = s * _SC_CHUNK
    pltpu.sync_copy(keys_hbm.at[pl.ds(base, _SC_CHUNK)], kv)
    lane = lax.iota(jnp.int32, 16)

    lo = jnp.zeros((16,), jnp.int32)
    for r, step in enumerate(_STEPS):
        thrs = [lo + jnp.int32((t + 1) * step) for t in range(15)]

        def vloop(i, accs, _thrs=thrs):
            k = kv[pl.ds(i * 16, 16)]
            return tuple(
                acc + jnp.where(k < th, 1, 0).astype(jnp.int32)
                for acc, th in zip(accs, _thrs))

        accs = lax.fori_loop(
            0, _SC_VREGS, vloop,
            tuple(jnp.zeros((16,), jnp.int32) for _ in range(15)))

        cnt = jnp.where(lane == jnp.int32(15), jnp.int32(1 << 20), jnp.int32(0))
        for t in range(15):
            cnt = jnp.where(lane == jnp.int32(t), _lane_sum(accs[t], lane), cnt)
        cnt_ref[...] = cnt
        pltpu.sync_copy(cnt_ref, shared.at[r, s])
        plsc.subcore_barrier()
        pltpu.sync_copy(shared.at[r], allc_ref)
        tot = allc_ref[0]
        for w in range(1, 16):
            tot = tot + allc_ref[w]
        tstar = _lane_sum(
            jnp.where(tot < jnp.int32(_K), 1, 0).astype(jnp.int32), lane)
        lo = lo + tstar * jnp.int32(step)

    def wloop(i, carry):
        k = kv[pl.ds(i * 16, 16)]
        ov[pl.ds(i * 16, 16)] = jnp.where(
            k <= lo, jnp.float32(1.0), jnp.float32(0.0))
        return carry

    lax.fori_loop(0, _SC_VREGS, wloop, jnp.int32(0))

    @pl.when(c == 0)
    def _():
        pltpu.sync_copy(ov, out_hbm.at[pl.ds(base, _SC_CHUNK)])


_sc_sample = functools.partial(
    pl.kernel,
    mesh=plsc.VectorSubcoreMesh(core_axis_name="c", subcore_axis_name="s",
                                num_cores=1),
    out_type=jax.ShapeDtypeStruct((_NPAD,), jnp.float32),
    scratch_types=[
        pltpu.VMEM((_SC_CHUNK,), jnp.int32),
        pltpu.VMEM((_SC_CHUNK,), jnp.float32),
        pltpu.VMEM((16,), jnp.int32),
        pltpu.VMEM((16, 16), jnp.int32),
        pltpu.VMEM_SHARED((4, 16, 16), jnp.int32),
    ],
)(_sc_sample_body)


def kernel(anchors, gt_boxes, gt_classes):
    pad = jnp.tile(jnp.asarray([[0.0, 0.0, 1.0, 1.0]], jnp.float32),
                   (_NPAD - _N, 1))
    a_t = jnp.concatenate([anchors, pad], axis=0).T.reshape(4, _ROWS, 128)
    boxes, posf, keys = _run_tc_match(a_t, gt_boxes, jnp.asarray(_key_bands()))
    ind = _sc_sample(keys.reshape(_NPAD))
    enc_t = _run_tc_encode(a_t, boxes)
    enc = enc_t.reshape(4, _NPAD)[:, :_N].T
    posc = posf.reshape(_NPAD)[:_N, None]
    cls_w = ind[:_N, None]
    return enc, posc, posc, cls_w


# speedup vs baseline: 1.2079x; 1.2079x over previous
import functools

import jax
import jax.numpy as jnp
import numpy as np
from jax import lax
from jax.experimental import pallas as pl
from jax.experimental.pallas import tpu as pltpu
from jax.experimental.pallas import tpu_sc as plsc

_N = 20000
_M = 100
_NPAD = 20480
_ROWS = _NPAD // 128
_K = 256
_SENT = np.int32(1 << 20)

def _rank_desc(v):
    order = np.argsort(-v, kind="stable")
    r = np.empty(_N, np.int32)
    r[order] = np.arange(_N, dtype=np.int32)
    return r


def _threefry2x32(k0, k1, x0, x1):
    rot = ((13, 15, 26, 6), (17, 29, 16, 24))
    ks = (np.uint32(k0), np.uint32(k1),
          np.uint32(np.uint32(k0) ^ np.uint32(k1) ^ np.uint32(0x1BD11BDA)))
    x0 = (x0 + ks[0]).astype(np.uint32)
    x1 = (x1 + ks[1]).astype(np.uint32)
    for i in range(5):
        for r in rot[i % 2]:
            x0 = (x0 + x1).astype(np.uint32)
            x1 = ((x1 << np.uint32(r)) | (x1 >> np.uint32(32 - r))).astype(np.uint32)
            x1 = (x1 ^ x0).astype(np.uint32)
        x0 = (x0 + ks[(i + 1) % 3]).astype(np.uint32)
        x1 = (x1 + ks[(i + 2) % 3] + np.uint32(i + 1)).astype(np.uint32)
    return x0, x1


def _draw_tiebreak():
    x0, x1 = _threefry2x32(0, 42, np.zeros(_N, np.uint32),
                           np.arange(_N, dtype=np.uint32))
    bits = (x0 ^ x1).astype(np.uint32)
    f = ((bits >> np.uint32(9)) | np.uint32(0x3F800000)).view(np.float32)
    u = (f - np.float32(1.0)) * np.float32(0.001) + np.float32(0.0)
    return np.maximum(np.float32(0.0), u)


def _build_key_bands():
    rnd = _draw_tiebreak()
    kpos = _rank_desc((np.float32(1.0) + rnd).astype(np.float32))
    kneg = _rank_desc((np.float32(0.5) + rnd).astype(np.float32)) + np.int32(_N)
    kinv = np.arange(_N, dtype=np.int32) + np.int32(2 * _N)

    def pad_band(b):
        return np.concatenate([b, np.full(_NPAD - _N, _SENT, np.int32)])

    return np.stack(
        [pad_band(kpos), pad_band(kneg), pad_band(kinv)]).reshape(3, _ROWS, 128)


_KBANDS = _build_key_bands()


def _key_bands():
    return _KBANDS


_CHUNK_ROWS = 32


def _tc_match_body(a_ref, gt_ref, kb_ref, box_ref, pos_ref, key_ref):
    for ch in range(_ROWS // _CHUNK_ROWS):
        sl = slice(ch * _CHUNK_ROWS, (ch + 1) * _CHUNK_ROWS)
        ay0 = a_ref[0, sl, :]
        ax0 = a_ref[1, sl, :]
        ay1 = a_ref[2, sl, :]
        ax1 = a_ref[3, sl, :]
        area_a = (ay1 - ay0) * (ax1 - ax0)

        def gt_step(j, carry, ay0=ay0, ax0=ax0, ay1=ay1, ax1=ax1,
                    area_a=area_a):
            bn, bd, by0, bx0, by1, bx1 = carry
            gy0 = gt_ref[j, 0]
            gx0 = gt_ref[j, 1]
            gy1 = gt_ref[j, 2]
            gx1 = gt_ref[j, 3]
            iy0 = jnp.maximum(ay0, gy0)
            ix0 = jnp.maximum(ax0, gx0)
            iy1 = jnp.minimum(ay1, gy1)
            ix1 = jnp.minimum(ax1, gx1)
            ih = jnp.maximum(iy1 - iy0, jnp.float32(0.0))
            iw = jnp.maximum(ix1 - ix0, jnp.float32(0.0))
            inter = ih * iw
            area_g = (gy1 - gy0) * (gx1 - gx0)
            union = (area_a + area_g) - inter
            upd = inter * bd > bn * union
            bn = jnp.where(upd, inter, bn)
            bd = jnp.where(upd, union, bd)
            by0 = jnp.where(upd, gy0, by0)
            bx0 = jnp.where(upd, gx0, bx0)
            by1 = jnp.where(upd, gy1, by1)
            bx1 = jnp.where(upd, gx1, bx1)
            return bn, bd, by0, bx0, by1, bx1

        shp = (_CHUNK_ROWS, 128)
        neg1 = jnp.full(shp, -1.0, jnp.float32)
        one = jnp.ones(shp, jnp.float32)
        zero = jnp.zeros(shp, jnp.float32)
        bn, bd, by0, bx0, by1, bx1 = lax.fori_loop(
            0, _M, gt_step, (neg1, one, zero, zero, zero, zero), unroll=4)

        best = bn / bd
        pos = best >= jnp.float32(0.7)
        neg = best < jnp.float32(0.3)
        pos_ref[sl, :] = jnp.where(pos, jnp.float32(1.0), jnp.float32(0.0))

        box_ref[0, sl, :] = by0
        box_ref[1, sl, :] = bx0
        box_ref[2, sl, :] = by1
        box_ref[3, sl, :] = bx1

        key_ref[sl, :] = jnp.where(
            pos, kb_ref[0, sl, :],
            jnp.where(neg, kb_ref[1, sl, :], kb_ref[2, sl, :]))


def _tc_encode_body(a_ref, box_ref, enc_ref):
    ay0 = a_ref[0]
    ax0 = a_ref[1]
    ay1 = a_ref[2]
    ax1 = a_ref[3]
    by0 = box_ref[0]
    bx0 = box_ref[1]
    by1 = box_ref[2]
    bx1 = box_ref[3]
    ah = ay1 - ay0
    aw = ax1 - ax0
    acy = ay0 + jnp.float32(0.5) * ah
    acx = ax0 + jnp.float32(0.5) * aw
    bh = by1 - by0
    bw = bx1 - bx0
    bcy = by0 + jnp.float32(0.5) * bh
    bcx = bx0 + jnp.float32(0.5) * bw
    enc_ref[0] = ((bcy - acy) / ah) / jnp.float32(0.1)
    enc_ref[1] = ((bcx - acx) / aw) / jnp.float32(0.1)
    enc_ref[2] = jnp.log(bh / ah) / jnp.float32(0.2)
    enc_ref[3] = jnp.log(bw / aw) / jnp.float32(0.2)


def _run_tc_match(a_t, gt_boxes, kbands):
    f32 = jnp.float32
    return pl.pallas_call(
        _tc_match_body,
        out_shape=[
            jax.ShapeDtypeStruct((4, _ROWS, 128), f32),
            jax.ShapeDtypeStruct((_ROWS, 128), f32),
            jax.ShapeDtypeStruct((_ROWS, 128), jnp.int32),
        ],
        in_specs=[
            pl.BlockSpec(memory_space=pltpu.VMEM),
            pl.BlockSpec(memory_space=pltpu.SMEM),
            pl.BlockSpec(memory_space=pltpu.VMEM),
        ],
        out_specs=[
            pl.BlockSpec(memory_space=pltpu.VMEM),
            pl.BlockSpec(memory_space=pltpu.VMEM),
            pl.BlockSpec(memory_space=pltpu.VMEM),
        ],
    )(a_t, gt_boxes, kbands)


def _run_tc_encode(a_t, boxes):
    return pl.pallas_call(
        _tc_encode_body,
        out_shape=jax.ShapeDtypeStruct((4, _ROWS, 128), jnp.float32),
        in_specs=[
            pl.BlockSpec(memory_space=pltpu.VMEM),
            pl.BlockSpec(memory_space=pltpu.VMEM),
        ],
        out_specs=pl.BlockSpec(memory_space=pltpu.VMEM),
    )(a_t, boxes)


_SC_CHUNK = _NPAD // 16
_SC_VREGS = _SC_CHUNK // 16
_STEPS = (4096, 256, 16, 1)


_GATHER_DNUMS = lax.GatherDimensionNumbers(
    offset_dims=(), collapsed_slice_dims=(0,), start_index_map=(0,))


def _perm16(x, idx):
    return lax.gather(x, idx.reshape(16, 1), _GATHER_DNUMS, (1,),
                      mode=lax.GatherScatterMode.PROMISE_IN_BOUNDS)


def _lane_sum(x, lane):
    for sh in (1, 2, 4, 8):
        x = x + _perm16(x, jnp.bitwise_and(lane + jnp.int32(sh), jnp.int32(15)))
    return x


def _sc_sample_body(keys_hbm, out_hbm, kv, ov, cnt_ref, allc_ref, shared):
    c = lax.axis_index("c")
    s = lax.axis_index("s")
    base = s * _SC_CHUNK
    pltpu.sync_copy(keys_hbm.at[pl.ds(base, _SC_CHUNK)], kv)
    lane = lax.iota(jnp.int32, 16)

    lo = jnp.zeros((16,), jnp.int32)
    for r, step in enumerate(_STEPS):
        thrs = [lo + jnp.int32((t + 1) * step) for t in range(15)]

        def vloop(i, accs, _thrs=thrs):
            k = kv[pl.ds(i * 16, 16)]
            return tuple(
                acc + jnp.where(k < th, 1, 0).astype(jnp.int32)
                for acc, th in zip(accs, _thrs))

        accs = lax.fori_loop(
            0, _SC_VREGS, vloop,
            tuple(jnp.zeros((16,), jnp.int32) for _ in range(15)))

        cnt = jnp.where(lane == jnp.int32(15), jnp.int32(1 << 20), jnp.int32(0))
        for t in range(15):
            cnt = jnp.where(lane == jnp.int32(t), _lane_sum(accs[t], lane), cnt)
        cnt_ref[...] = cnt
        pltpu.sync_copy(cnt_ref, shared.at[r, s])
        plsc.subcore_barrier()
        pltpu.sync_copy(shared.at[r], allc_ref)
        tot = allc_ref[0]
        for w in range(1, 16):
            tot = tot + allc_ref[w]
        tstar = _lane_sum(
            jnp.where(tot < jnp.int32(_K), 1, 0).astype(jnp.int32), lane)
        lo = lo + tstar * jnp.int32(step)

    def wloop(i, carry):
        k = kv[pl.ds(i * 16, 16)]
        ov[pl.ds(i * 16, 16)] = jnp.where(
            k <= lo, jnp.float32(1.0), jnp.float32(0.0))
        return carry

    lax.fori_loop(0, _SC_VREGS, wloop, jnp.int32(0))

    @pl.when(c == 0)
    def _():
        pltpu.sync_copy(ov, out_hbm.at[pl.ds(base, _SC_CHUNK)])


_sc_sample = functools.partial(
    pl.kernel,
    mesh=plsc.VectorSubcoreMesh(core_axis_name="c", subcore_axis_name="s",
                                num_cores=1),
    out_type=jax.ShapeDtypeStruct((_NPAD,), jnp.float32),
    scratch_types=[
        pltpu.VMEM((_SC_CHUNK,), jnp.int32),
        pltpu.VMEM((_SC_CHUNK,), jnp.float32),
        pltpu.VMEM((16,), jnp.int32),
        pltpu.VMEM((16, 16), jnp.int32),
        pltpu.VMEM_SHARED((4, 16, 16), jnp.int32),
    ],
)(_sc_sample_body)


def kernel(anchors, gt_boxes, gt_classes):
    pad = jnp.tile(jnp.asarray([[0.0, 0.0, 1.0, 1.0]], jnp.float32),
                   (_NPAD - _N, 1))
    a_t = jnp.concatenate([anchors, pad], axis=0).T.reshape(4, _ROWS, 128)
    boxes, posf, keys = _run_tc_match(a_t, gt_boxes, jnp.asarray(_key_bands()))
    ind = _sc_sample(keys.reshape(_NPAD))
    enc_t = _run_tc_encode(a_t, boxes)
    enc = enc_t.reshape(4, _NPAD)[:, :_N].T
    posc = posf.reshape(_NPAD)[:_N, None]
    cls_w = ind[:_N, None]
    return enc, posc, posc, cls_w
